# baseline (speedup 1.0000x reference)
import jax
import jax.numpy as jnp
from jax.experimental import pallas as pl
from jax.experimental.pallas import tpu as pltpu

_NK = 1000
_NSPLIT = 8


def _mv_cols(*refs):
    x_refs = refs[:_NSPLIT]
    st_ref, ind_ref, joint_ref, s_scr = refs[_NSPLIT:]
    k = pl.program_id(0)
    nb = ind_ref.shape[1] // _NSPLIT

    @pl.when(k == 0)
    def _prep():
        s_scr[...] = st_ref[...].T
        joint_ref[...] = jnp.zeros_like(joint_ref)

    s = s_scr[pl.ds(k * _NK, _NK), :]
    for j, xr in enumerate(x_refs):
        part = jnp.sum(xr[...] * s, axis=0, keepdims=True)
        sl = (slice(0, 1), slice(j * nb, (j + 1) * nb))

        @pl.when(k == 0)
        def _init(part=part, sl=sl):
            ind_ref[sl] = part

        @pl.when(k > 0)
        def _acc(part=part, sl=sl):
            ind_ref[sl] += part


def kernel(xs_0, xs_1, xs_2, xs_3, layer_tab, type_tab, mod_tab, score,
           W1, b1, W2, b2):
    B, NP = xs_3.shape
    xt = xs_3.T
    st = score.T
    nb = B // _NSPLIT
    x_specs = [
        pl.BlockSpec((_NK, nb), lambda k, j=j: (k, j))
        for j in range(_NSPLIT)
    ]
    individual, joint = pl.pallas_call(
        _mv_cols,
        grid=(NP // _NK,),
        in_specs=x_specs + [pl.BlockSpec((1, NP), lambda k: (0, 0))],
        out_specs=[
            pl.BlockSpec((1, B), lambda k: (0, 0)),
            pl.BlockSpec((1, B), lambda k: (0, 0)),
        ],
        out_shape=[
            jax.ShapeDtypeStruct((1, B), jnp.float32),
            jax.ShapeDtypeStruct((1, B), jnp.float32),
        ],
        scratch_shapes=[pltpu.VMEM((NP, 1), jnp.float32)],
        compiler_params=pltpu.CompilerParams(
            dimension_semantics=("arbitrary",),
            vmem_limit_bytes=48 * 1024 * 1024),
    )(*([xt] * _NSPLIT), st)
    return (individual.reshape(B, 1), joint.reshape(B, 1))


# speedup vs baseline: 1.0131x; 1.0131x over previous
import jax
import jax.numpy as jnp
from jax.experimental import pallas as pl
from jax.experimental.pallas import tpu as pltpu

_NK = 1000
_NSPLIT = 4


def _mv_cols(*refs):
    x_refs = refs[:_NSPLIT]
    st_ref, ind_ref, joint_ref, s_scr = refs[_NSPLIT:]
    k = pl.program_id(0)
    nb = ind_ref.shape[1] // _NSPLIT

    @pl.when(k == 0)
    def _prep():
        s_scr[...] = st_ref[...].T
        joint_ref[...] = jnp.zeros_like(joint_ref)

    s = s_scr[pl.ds(k * _NK, _NK), :]
    for j, xr in enumerate(x_refs):
        part = jnp.sum(xr[...] * s, axis=0, keepdims=True)
        sl = (slice(0, 1), slice(j * nb, (j + 1) * nb))

        @pl.when(k == 0)
        def _init(part=part, sl=sl):
            ind_ref[sl] = part

        @pl.when(k > 0)
        def _acc(part=part, sl=sl):
            ind_ref[sl] += part


def kernel(xs_0, xs_1, xs_2, xs_3, layer_tab, type_tab, mod_tab, score,
           W1, b1, W2, b2):
    B, NP = xs_3.shape
    xt = xs_3.T
    st = score.T
    nb = B // _NSPLIT
    x_specs = [
        pl.BlockSpec((_NK, nb), lambda k, j=j: (k, j))
        for j in range(_NSPLIT)
    ]
    individual, joint = pl.pallas_call(
        _mv_cols,
        grid=(NP // _NK,),
        in_specs=x_specs + [pl.BlockSpec((1, NP), lambda k: (0, 0))],
        out_specs=[
            pl.BlockSpec((1, B), lambda k: (0, 0)),
            pl.BlockSpec((1, B), lambda k: (0, 0)),
        ],
        out_shape=[
            jax.ShapeDtypeStruct((1, B), jnp.float32),
            jax.ShapeDtypeStruct((1, B), jnp.float32),
        ],
        scratch_shapes=[pltpu.VMEM((NP, 1), jnp.float32)],
        compiler_params=pltpu.CompilerParams(
            dimension_semantics=("arbitrary",),
            vmem_limit_bytes=48 * 1024 * 1024),
    )(*([xt] * _NSPLIT), st)
    return (individual.reshape(B, 1), joint.reshape(B, 1))
